# baseline (speedup 1.0000x reference)
import jax
import jax.numpy as jnp
from jax import lax
from jax.experimental import pallas as pl
from jax.experimental.pallas import tpu as pltpu
from jax.experimental.pallas import tpu_sc as plsc

N_NODES = 100000
N_EDGES = 1600000
PAD_NODE = N_NODES
N_P = 100352
E_P = 1638400
CH = 128
GRP = 5
NC, NS = 2, 16
ROWS_PER_STAGE = 20
GPS = ROWS_PER_STAGE // GRP
RW_SC0 = 600
RW_SC1 = 200
STAGES0 = RW_SC0 // ROWS_PER_STAGE
STAGES1 = RW_SC1 // ROWS_PER_STAGE
ACC_N = 100016
ACC_ROWS_PER_TILE = ACC_N // NS
NZ = 47
ZCH = ACC_ROWS_PER_TILE // NZ


def _seg_body_multi(*refs):
    ntab = len(refs) - 12
    tables = refs[:ntab]
    (src_hbm, dst_hbm, out_hbm, acc_sh, rv0, rv1, svb, dvb, zbuf,
     gsem0, gsem1, ssem) = refs[ntab:]
    c = lax.axis_index("c")
    s = lax.axis_index("s")
    z0 = s * ACC_ROWS_PER_TILE
    wbase = jnp.where(c == 0, s * RW_SC0, NS * RW_SC0 + s * RW_SC1)
    nstages = jnp.where(c == 0, STAGES0, STAGES1)

    def _zfill(i, _):
        zbuf[i, :] = jnp.zeros((16,), jnp.float32)
        return 0
    lax.fori_loop(0, ZCH, _zfill, 0)

    for p, table in enumerate(tables):
        zc = [pltpu.async_copy(zbuf, acc_sh.at[pl.ds(z0 + k * ZCH, ZCH)],
                               gsem0) for k in range(NZ)]
        for d in zc:
            d.wait()
        plsc.subcore_barrier()

        def _stage(f, _, table=table):
            fb = wbase + f * ROWS_PER_STAGE
            pltpu.sync_copy(src_hbm.at[pl.ds(fb, ROWS_PER_STAGE)], svb)
            pltpu.sync_copy(dst_hbm.at[pl.ds(fb, ROWS_PER_STAGE)], dvb)
            for j in range(GRP):
                pltpu.async_copy(table.at[svb.at[j]], rv0.at[j], gsem0)

            def _pair(i, _):
                r1 = (2 * i + 1) * GRP
                for j in range(GRP):
                    pltpu.async_copy(table.at[svb.at[r1 + j]],
                                     rv1.at[j], gsem1)
                for j in range(GRP):
                    pltpu.make_async_copy(table.at[svb.at[j]],
                                          rv0.at[j], gsem0).wait()
                r0 = (2 * i) * GRP
                sc0 = [pltpu.async_copy(rv0.at[j],
                                        acc_sh.at[dvb.at[r0 + j]],
                                        ssem, add=True)
                       for j in range(GRP)]
                for d in sc0:
                    d.wait()
                r2 = ((2 * i + 2) * GRP) % ROWS_PER_STAGE
                for j in range(GRP):
                    pltpu.async_copy(table.at[svb.at[r2 + j]],
                                     rv0.at[j], gsem0)
                for j in range(GRP):
                    pltpu.make_async_copy(table.at[svb.at[j]],
                                          rv1.at[j], gsem1).wait()
                sc1 = [pltpu.async_copy(rv1.at[j],
                                        acc_sh.at[dvb.at[r1 + j]],
                                        ssem, add=True)
                       for j in range(GRP)]
                for d in sc1:
                    d.wait()
                return 0

            lax.fori_loop(0, GPS // 2, _pair, 0)
            for j in range(GRP):
                pltpu.make_async_copy(table.at[svb.at[j]],
                                      rv0.at[j], gsem0).wait()
            return 0

        lax.fori_loop(0, nstages, _stage, 0)
        plsc.subcore_barrier()
        pltpu.sync_copy(acc_sh.at[pl.ds(z0, ACC_ROWS_PER_TILE)],
                        out_hbm.at[p, c, pl.ds(z0, ACC_ROWS_PER_TILE)])


def _seg_sum16(tables, src_rows, dst_rows):
    ntab = len(tables)
    return pl.kernel(
        _seg_body_multi,
        out_type=jax.ShapeDtypeStruct((ntab, NC, N_P, 16), jnp.float32),
        mesh=plsc.VectorSubcoreMesh(core_axis_name="c", subcore_axis_name="s"),
        compiler_params=pltpu.CompilerParams(use_tc_tiling_on_sc=False),
        scratch_types=[
            pltpu.VMEM_SHARED((ACC_N, 16), jnp.float32),
            pltpu.VMEM((GRP, CH, 16), jnp.float32),
            pltpu.VMEM((GRP, CH, 16), jnp.float32),
            pltpu.VMEM((ROWS_PER_STAGE, CH), jnp.int32),
            pltpu.VMEM((ROWS_PER_STAGE, CH), jnp.int32),
            pltpu.VMEM((ZCH, 16), jnp.float32),
            pltpu.SemaphoreType.DMA,
            pltpu.SemaphoreType.DMA,
            pltpu.SemaphoreType.DMA,
        ],
    )(*tables, src_rows, dst_rows)


BLK = 2048
GRID = N_P // BLK


def _bs(f):
    return pl.BlockSpec((BLK, f), lambda i: (i, 0))


def _full(shape):
    return pl.BlockSpec(shape, lambda i: tuple(0 for _ in shape))


def _aspec(p):
    return pl.BlockSpec((None, 2, BLK, 16), lambda i, p=p: (p, 0, i, 0))


def _l1_body(a, x, Wl, Wr, b, h0, h1, h2, h3, invd):
    acc = a[0] + a[1]
    inv = 1.0 / jnp.maximum(acc[:, 4:5], 1.0)
    h = jnp.maximum(
        jnp.dot(acc[:, :4] * inv, Wl[...], preferred_element_type=jnp.float32)
        + jnp.dot(x[...], Wr[...], preferred_element_type=jnp.float32)
        + b[...], 0.0)
    h0[...] = h[:, 0:16]
    h1[...] = h[:, 16:32]
    h2[...] = h[:, 32:48]
    h3[...] = h[:, 48:64]
    invd[...] = inv


def _tc_layer1(acc, x4, Wl1, Wr1, b1):
    return pl.pallas_call(
        _l1_body,
        grid=(GRID,),
        in_specs=[_aspec(0), _bs(4), _full((4, 64)), _full((4, 64)),
                  _full((1, 64))],
        out_specs=[_bs(16), _bs(16), _bs(16), _bs(16), _bs(1)],
        out_shape=[jax.ShapeDtypeStruct((N_P, 16), jnp.float32)] * 4
        + [jax.ShapeDtypeStruct((N_P, 1), jnp.float32)],
    )(acc, x4, Wl1, Wr1, b1)


def _l2_body(a0, a1, a2, a3, h0, h1, h2, h3, invd, Wl, Wr, b, Wl3,
             hout, m0, m1):
    agg = jnp.concatenate(
        [a0[0] + a0[1], a1[0] + a1[1], a2[0] + a2[1], a3[0] + a3[1]], axis=1)
    hprev = jnp.concatenate([h0[...], h1[...], h2[...], h3[...]], axis=1)
    h = jnp.maximum(
        jnp.dot(agg * invd[...], Wl[...], preferred_element_type=jnp.float32)
        + jnp.dot(hprev, Wr[...], preferred_element_type=jnp.float32)
        + b[...], 0.0)
    m = jnp.dot(h, Wl3[...], preferred_element_type=jnp.float32)
    hout[...] = h
    m0[...] = m[:, 0:16]
    m1[...] = m[:, 16:32]


def _tc_layer2(accs, hs, invd, Wl2, Wr2, b2, Wl3):
    return pl.pallas_call(
        _l2_body,
        grid=(GRID,),
        in_specs=[_aspec(0), _aspec(1), _aspec(2), _aspec(3)]
        + [_bs(16)] * 4 + [_bs(1)]
        + [_full((64, 64)), _full((64, 64)), _full((1, 64)),
           _full((64, 32))],
        out_specs=[_bs(64), _bs(16), _bs(16)],
        out_shape=[jax.ShapeDtypeStruct((N_P, 64), jnp.float32),
                   jax.ShapeDtypeStruct((N_P, 16), jnp.float32),
                   jax.ShapeDtypeStruct((N_P, 16), jnp.float32)],
    )(accs, accs, accs, accs, *hs, invd, Wl2, Wr2, b2, Wl3)


def _l3_body(a0, a1, h2, invd, Wr, b, W4, b4, W5, b5, out):
    agg = jnp.concatenate([a0[0] + a0[1], a1[0] + a1[1]], axis=1)
    h = jnp.maximum(
        agg * invd[...]
        + jnp.dot(h2[...], Wr[...], preferred_element_type=jnp.float32)
        + b[...], 0.0)
    h = jnp.maximum(
        jnp.dot(h, W4[...], preferred_element_type=jnp.float32)
        + b4[...], 0.0)
    out[...] = (jnp.dot(h, W5[...], preferred_element_type=jnp.float32)
                + b5[...])


def _tc_layer3(accs, h2, invd, Wr3, b3, W4, b4, W5, b5):
    return pl.pallas_call(
        _l3_body,
        grid=(GRID,),
        in_specs=[_aspec(0), _aspec(1), _bs(64), _bs(1),
                  _full((64, 32)), _full((1, 32)), _full((32, 16)),
                  _full((1, 16)), _full((16, 1)), _full((1, 1))],
        out_specs=_bs(1),
        out_shape=jax.ShapeDtypeStruct((N_P, 1), jnp.float32),
    )(accs, accs, h2, invd, Wr3, b3, W4, b4, W5, b5)


def kernel(x, edge_index, Wl1, Wr1, b1, Wl2, Wr2, b2, Wl3, Wr3, b3,
           W4, b4, W5, b5):
    src = edge_index[0].astype(jnp.int32)
    dst = edge_index[1].astype(jnp.int32)
    padi = jnp.full((E_P - N_EDGES,), PAD_NODE, jnp.int32)
    src_rows = jnp.concatenate([src, padi]).reshape(E_P // CH, CH)
    dst_rows = jnp.concatenate([dst, padi]).reshape(E_P // CH, CH)

    ones = jnp.ones((N_NODES, 1), jnp.float32)
    x16 = jnp.pad(jnp.concatenate([x, ones], axis=1),
                  ((0, N_P - N_NODES), (0, 11)))
    x4 = jnp.pad(x, ((0, N_P - N_NODES), (0, 0)))

    acc1 = _seg_sum16([x16], src_rows, dst_rows)
    h0, h1, h2t, h3t, invd = _tc_layer1(acc1, x4, Wl1.astype(jnp.float32),
                                        Wr1, b1.reshape(1, 64))

    a2 = _seg_sum16([h0, h1, h2t, h3t], src_rows, dst_rows)
    h2, m0, m1 = _tc_layer2(a2, (h0, h1, h2t, h3t), invd,
                            Wl2, Wr2, b2.reshape(1, 64), Wl3)

    a3 = _seg_sum16([m0, m1], src_rows, dst_rows)
    out = _tc_layer3(a3, h2, invd, Wr3, b3.reshape(1, 32),
                     W4, b4.reshape(1, 16), W5, b5.reshape(1, 1))
    return out[:N_NODES, 0]


# speedup vs baseline: 1.2001x; 1.2001x over previous
import jax
import jax.numpy as jnp
from jax import lax
from jax.experimental import pallas as pl
from jax.experimental.pallas import tpu as pltpu
from jax.experimental.pallas import tpu_sc as plsc

N_NODES = 100000
N_EDGES = 1600000
PAD_NODE = N_NODES
N_P = 100352
E_P = 1638400
CH = 128
GRP = 4
NC, NS = 2, 16
ROWS_PER_STAGE = 40
GPS = ROWS_PER_STAGE // GRP
ROWS_PER_TILE = E_P // CH // NS
STAGES = ROWS_PER_TILE // ROWS_PER_STAGE
ACC_N = 100016
ACC_ROWS_PER_TILE = ACC_N // NS
NZ = 47
ZCH = ACC_ROWS_PER_TILE // NZ


def _make_seg_body(assign):
  def _seg_body_multi(*refs):
    ntab = len(refs) - 12
    tables = refs[:ntab]
    (src_hbm, dst_hbm, out_hbm, acc_sh, rv0, rv1, svb, dvb, zbuf,
     gsem0, gsem1, ssem) = refs[ntab:]
    c = lax.axis_index("c")
    s = lax.axis_index("s")
    z0 = s * ACC_ROWS_PER_TILE
    wbase = s * ROWS_PER_TILE

    def _zfill(i, _):
        zbuf[i, :] = jnp.zeros((16,), jnp.float32)
        return 0
    lax.fori_loop(0, ZCH, _zfill, 0)

    for p, (table, core) in enumerate(zip(tables, assign)):
      @pl.when(c == core)
      def _panel(p=p, table=table):
        zc = [pltpu.async_copy(zbuf, acc_sh.at[pl.ds(z0 + k * ZCH, ZCH)],
                               gsem0) for k in range(NZ)]
        for d in zc:
            d.wait()
        plsc.subcore_barrier()

        def _stage(f, _, table=table):
            fb = wbase + f * ROWS_PER_STAGE
            pltpu.sync_copy(src_hbm.at[pl.ds(fb, ROWS_PER_STAGE)], svb)
            pltpu.sync_copy(dst_hbm.at[pl.ds(fb, ROWS_PER_STAGE)], dvb)
            for j in range(GRP):
                pltpu.async_copy(table.at[svb.at[j]], rv0.at[j], gsem0)

            def _pair(i, _):
                r1 = (2 * i + 1) * GRP
                for j in range(GRP):
                    pltpu.async_copy(table.at[svb.at[r1 + j]],
                                     rv1.at[j], gsem1)
                for j in range(GRP):
                    pltpu.make_async_copy(table.at[svb.at[j]],
                                          rv0.at[j], gsem0).wait()
                r0 = (2 * i) * GRP
                sc0 = [pltpu.async_copy(rv0.at[j],
                                        acc_sh.at[dvb.at[r0 + j]],
                                        ssem, add=True)
                       for j in range(GRP)]
                for d in sc0:
                    d.wait()
                r2 = ((2 * i + 2) * GRP) % ROWS_PER_STAGE
                for j in range(GRP):
                    pltpu.async_copy(table.at[svb.at[r2 + j]],
                                     rv0.at[j], gsem0)
                for j in range(GRP):
                    pltpu.make_async_copy(table.at[svb.at[j]],
                                          rv1.at[j], gsem1).wait()
                sc1 = [pltpu.async_copy(rv1.at[j],
                                        acc_sh.at[dvb.at[r1 + j]],
                                        ssem, add=True)
                       for j in range(GRP)]
                for d in sc1:
                    d.wait()
                return 0

            lax.fori_loop(0, GPS // 2, _pair, 0)
            for j in range(GRP):
                pltpu.make_async_copy(table.at[svb.at[j]],
                                      rv0.at[j], gsem0).wait()
            return 0

        lax.fori_loop(0, STAGES, _stage, 0)
        plsc.subcore_barrier()
        pltpu.sync_copy(acc_sh.at[pl.ds(z0, ACC_ROWS_PER_TILE)],
                        out_hbm.at[p, pl.ds(z0, ACC_ROWS_PER_TILE)])

  return _seg_body_multi


def _seg_sum16(tables, src_rows, dst_rows, assign):
    ntab = len(tables)
    return pl.kernel(
        _make_seg_body(assign),
        out_type=jax.ShapeDtypeStruct((ntab, N_P, 16), jnp.float32),
        mesh=plsc.VectorSubcoreMesh(core_axis_name="c", subcore_axis_name="s"),
        compiler_params=pltpu.CompilerParams(use_tc_tiling_on_sc=False),
        scratch_types=[
            pltpu.VMEM_SHARED((ACC_N, 16), jnp.float32),
            pltpu.VMEM((GRP, CH, 16), jnp.float32),
            pltpu.VMEM((GRP, CH, 16), jnp.float32),
            pltpu.VMEM((ROWS_PER_STAGE, CH), jnp.int32),
            pltpu.VMEM((ROWS_PER_STAGE, CH), jnp.int32),
            pltpu.VMEM((ZCH, 16), jnp.float32),
            pltpu.SemaphoreType.DMA,
            pltpu.SemaphoreType.DMA,
            pltpu.SemaphoreType.DMA,
        ],
    )(*tables, src_rows, dst_rows)


BLK = 2048
GRID = N_P // BLK


def _bs(f):
    return pl.BlockSpec((BLK, f), lambda i: (i, 0))


def _full(shape):
    return pl.BlockSpec(shape, lambda i: tuple(0 for _ in shape))


def _aspec(p):
    return pl.BlockSpec((None, BLK, 16), lambda i, p=p: (p, i, 0))


def _l1_body(a, x, Wl, Wr, b, h0, h1, h2, h3, invd):
    acc = a[...]
    inv = 1.0 / jnp.maximum(acc[:, 4:5], 1.0)
    h = jnp.maximum(
        jnp.dot(acc[:, :4] * inv, Wl[...], preferred_element_type=jnp.float32)
        + jnp.dot(x[...], Wr[...], preferred_element_type=jnp.float32)
        + b[...], 0.0)
    h0[...] = h[:, 0:16]
    h1[...] = h[:, 16:32]
    h2[...] = h[:, 32:48]
    h3[...] = h[:, 48:64]
    invd[...] = inv


def _tc_layer1(acc, x4, Wl1, Wr1, b1):
    return pl.pallas_call(
        _l1_body,
        grid=(GRID,),
        in_specs=[_aspec(0), _bs(4), _full((4, 64)), _full((4, 64)),
                  _full((1, 64))],
        out_specs=[_bs(16), _bs(16), _bs(16), _bs(16), _bs(1)],
        out_shape=[jax.ShapeDtypeStruct((N_P, 16), jnp.float32)] * 4
        + [jax.ShapeDtypeStruct((N_P, 1), jnp.float32)],
    )(acc, x4, Wl1, Wr1, b1)


def _l2_body(a0, a1, a2, a3, h0, h1, h2, h3, invd, Wl, Wr, b, Wl3,
             hout, m0, m1):
    agg = jnp.concatenate([a0[...], a1[...], a2[...], a3[...]], axis=1)
    hprev = jnp.concatenate([h0[...], h1[...], h2[...], h3[...]], axis=1)
    h = jnp.maximum(
        jnp.dot(agg * invd[...], Wl[...], preferred_element_type=jnp.float32)
        + jnp.dot(hprev, Wr[...], preferred_element_type=jnp.float32)
        + b[...], 0.0)
    m = jnp.dot(h, Wl3[...], preferred_element_type=jnp.float32)
    hout[...] = h
    m0[...] = m[:, 0:16]
    m1[...] = m[:, 16:32]


def _tc_layer2(accs, hs, invd, Wl2, Wr2, b2, Wl3):
    return pl.pallas_call(
        _l2_body,
        grid=(GRID,),
        in_specs=[_aspec(0), _aspec(1), _aspec(2), _aspec(3)]
        + [_bs(16)] * 4 + [_bs(1)]
        + [_full((64, 64)), _full((64, 64)), _full((1, 64)),
           _full((64, 32))],
        out_specs=[_bs(64), _bs(16), _bs(16)],
        out_shape=[jax.ShapeDtypeStruct((N_P, 64), jnp.float32),
                   jax.ShapeDtypeStruct((N_P, 16), jnp.float32),
                   jax.ShapeDtypeStruct((N_P, 16), jnp.float32)],
    )(accs, accs, accs, accs, *hs, invd, Wl2, Wr2, b2, Wl3)


def _l3_body(a0, a1, h2, invd, Wr, b, W4, b4, W5, b5, out):
    agg = jnp.concatenate([a0[...], a1[...]], axis=1)
    h = jnp.maximum(
        agg * invd[...]
        + jnp.dot(h2[...], Wr[...], preferred_element_type=jnp.float32)
        + b[...], 0.0)
    h = jnp.maximum(
        jnp.dot(h, W4[...], preferred_element_type=jnp.float32)
        + b4[...], 0.0)
    out[...] = (jnp.dot(h, W5[...], preferred_element_type=jnp.float32)
                + b5[...])


def _tc_layer3(accs, h2, invd, Wr3, b3, W4, b4, W5, b5):
    return pl.pallas_call(
        _l3_body,
        grid=(GRID,),
        in_specs=[_aspec(0), _aspec(1), _bs(64), _bs(1),
                  _full((64, 32)), _full((1, 32)), _full((32, 16)),
                  _full((1, 16)), _full((16, 1)), _full((1, 1))],
        out_specs=_bs(1),
        out_shape=jax.ShapeDtypeStruct((N_P, 1), jnp.float32),
    )(accs, accs, h2, invd, Wr3, b3, W4, b4, W5, b5)


def kernel(x, edge_index, Wl1, Wr1, b1, Wl2, Wr2, b2, Wl3, Wr3, b3,
           W4, b4, W5, b5):
    src = edge_index[0].astype(jnp.int32)
    dst = edge_index[1].astype(jnp.int32)
    padi = jnp.full((E_P - N_EDGES,), PAD_NODE, jnp.int32)
    src_rows = jnp.concatenate([src, padi]).reshape(E_P // CH, CH)
    dst_rows = jnp.concatenate([dst, padi]).reshape(E_P // CH, CH)

    ones = jnp.ones((N_NODES, 1), jnp.float32)
    x16 = jnp.pad(jnp.concatenate([x, ones], axis=1),
                  ((0, N_P - N_NODES), (0, 11)))
    x4 = jnp.pad(x, ((0, N_P - N_NODES), (0, 0)))

    acc1 = _seg_sum16([x16], src_rows, dst_rows, [0])
    h0, h1, h2t, h3t, invd = _tc_layer1(acc1, x4, Wl1.astype(jnp.float32),
                                        Wr1, b1.reshape(1, 64))

    a2 = _seg_sum16([h0, h1, h2t, h3t], src_rows, dst_rows, [0, 0, 0, 1])
    h2, m0, m1 = _tc_layer2(a2, (h0, h1, h2t, h3t), invd,
                            Wl2, Wr2, b2.reshape(1, 64), Wl3)

    a3 = _seg_sum16([m0, m1], src_rows, dst_rows, [0, 1])
    out = _tc_layer3(a3, h2, invd, Wr3, b3.reshape(1, 32),
                     W4, b4.reshape(1, 16), W5, b5.reshape(1, 1))
    return out[:N_NODES, 0]


# speedup vs baseline: 1.3327x; 1.1105x over previous
import jax
import jax.numpy as jnp
from jax import lax
from jax.experimental import pallas as pl
from jax.experimental.pallas import tpu as pltpu
from jax.experimental.pallas import tpu_sc as plsc

N_NODES = 100000
N_EDGES = 1600000
PAD_NODE = N_NODES
N_P = 100352
E_P = 1638400
CH = 128
GRP = 4
NC, NS = 2, 16
ROWS_PER_STAGE = 40
GPS = ROWS_PER_STAGE // GRP
ROWS_PER_TILE = E_P // CH // NS
STAGES = ROWS_PER_TILE // ROWS_PER_STAGE
ACC_N = 100016
ACC_ROWS_PER_TILE = ACC_N // NS
NZ = 47
ZCH = ACC_ROWS_PER_TILE // NZ


def _make_seg_body(assign):
  def _seg_body_multi(*refs):
    ntab = len(refs) - 12
    tables = refs[:ntab]
    (src_hbm, dst_hbm, out_hbm, acc_sh, rv0, rv1, svb, dvb, zbuf,
     gsem0, gsem1, ssem) = refs[ntab:]
    c = lax.axis_index("c")
    s = lax.axis_index("s")
    z0 = s * ACC_ROWS_PER_TILE
    wbase = s * ROWS_PER_TILE

    def _zfill(i, _):
        zbuf[i, :] = jnp.zeros((16,), jnp.float32)
        return 0
    lax.fori_loop(0, ZCH, _zfill, 0)

    for p, (table, core) in enumerate(zip(tables, assign)):
      @pl.when(c == core)
      def _panel(p=p, table=table):
        zc = [pltpu.async_copy(zbuf, acc_sh.at[pl.ds(z0 + k * ZCH, ZCH)],
                               gsem0) for k in range(NZ)]
        for d in zc:
            d.wait()
        plsc.subcore_barrier()

        def _stage(f, _, table=table):
            fb = wbase + f * ROWS_PER_STAGE
            pltpu.sync_copy(src_hbm.at[pl.ds(fb, ROWS_PER_STAGE)], svb)
            pltpu.sync_copy(dst_hbm.at[pl.ds(fb, ROWS_PER_STAGE)], dvb)
            for j in range(GRP):
                pltpu.async_copy(table.at[svb.at[j]], rv0.at[j], gsem0)

            def _pair(i, _):
                r1 = (2 * i + 1) * GRP
                for j in range(GRP):
                    pltpu.async_copy(table.at[svb.at[r1 + j]],
                                     rv1.at[j], gsem1)
                for j in range(GRP):
                    pltpu.make_async_copy(table.at[svb.at[j]],
                                          rv0.at[j], gsem0).wait()
                r0 = (2 * i) * GRP
                sc0 = [pltpu.async_copy(rv0.at[j],
                                        acc_sh.at[dvb.at[r0 + j]],
                                        ssem, add=True)
                       for j in range(GRP)]
                for d in sc0:
                    d.wait()
                r2 = ((2 * i + 2) * GRP) % ROWS_PER_STAGE
                for j in range(GRP):
                    pltpu.async_copy(table.at[svb.at[r2 + j]],
                                     rv0.at[j], gsem0)
                for j in range(GRP):
                    pltpu.make_async_copy(table.at[svb.at[j]],
                                          rv1.at[j], gsem1).wait()
                sc1 = [pltpu.async_copy(rv1.at[j],
                                        acc_sh.at[dvb.at[r1 + j]],
                                        ssem, add=True)
                       for j in range(GRP)]
                for d in sc1:
                    d.wait()
                return 0

            lax.fori_loop(0, GPS // 2, _pair, 0)
            for j in range(GRP):
                pltpu.make_async_copy(table.at[svb.at[j]],
                                      rv0.at[j], gsem0).wait()
            return 0

        lax.fori_loop(0, STAGES, _stage, 0)
        plsc.subcore_barrier()
        pltpu.sync_copy(acc_sh.at[pl.ds(z0, ACC_ROWS_PER_TILE)],
                        out_hbm.at[p, pl.ds(z0, ACC_ROWS_PER_TILE)])

  return _seg_body_multi


def _seg_sum16(tables, src_rows, dst_rows, assign):
    ntab = len(tables)
    return pl.kernel(
        _make_seg_body(assign),
        out_type=jax.ShapeDtypeStruct((ntab, N_P, 16), jnp.float32),
        mesh=plsc.VectorSubcoreMesh(core_axis_name="c", subcore_axis_name="s"),
        compiler_params=pltpu.CompilerParams(use_tc_tiling_on_sc=False),
        scratch_types=[
            pltpu.VMEM_SHARED((ACC_N, 16), jnp.float32),
            pltpu.VMEM((GRP, CH, 16), jnp.float32),
            pltpu.VMEM((GRP, CH, 16), jnp.float32),
            pltpu.VMEM((ROWS_PER_STAGE, CH), jnp.int32),
            pltpu.VMEM((ROWS_PER_STAGE, CH), jnp.int32),
            pltpu.VMEM((ZCH, 16), jnp.float32),
            pltpu.SemaphoreType.DMA,
            pltpu.SemaphoreType.DMA,
            pltpu.SemaphoreType.DMA,
        ],
    )(*tables, src_rows, dst_rows)


BLK = 2048
GRID = N_P // BLK


def _bs(f):
    return pl.BlockSpec((BLK, f), lambda i: (i, 0))


def _full(shape):
    return pl.BlockSpec(shape, lambda i: tuple(0 for _ in shape))


def _aspec(p):
    return pl.BlockSpec((None, BLK, 16), lambda i, p=p: (p, i, 0))


def _l1_body(a, x, Wl, Wr, b, h0, h1, h2, h3, invd):
    acc = a[...]
    inv = 1.0 / jnp.maximum(acc[:, 4:5], 1.0)
    h = jnp.maximum(
        jnp.dot(acc[:, :4] * inv, Wl[...], preferred_element_type=jnp.float32)
        + jnp.dot(x[...], Wr[...], preferred_element_type=jnp.float32)
        + b[...], 0.0)
    h0[...] = h[:, 0:16]
    h1[...] = h[:, 16:32]
    h2[...] = h[:, 32:48]
    h3[...] = h[:, 48:64]
    invd[...] = inv


def _tc_layer1(acc, x4, Wl1, Wr1, b1):
    return pl.pallas_call(
        _l1_body,
        grid=(GRID,),
        in_specs=[_aspec(0), _bs(4), _full((4, 64)), _full((4, 64)),
                  _full((1, 64))],
        out_specs=[_bs(16), _bs(16), _bs(16), _bs(16), _bs(1)],
        out_shape=[jax.ShapeDtypeStruct((N_P, 16), jnp.float32)] * 4
        + [jax.ShapeDtypeStruct((N_P, 1), jnp.float32)],
    )(acc, x4, Wl1, Wr1, b1)


def _l2_body(a0, a1, a2, a3, h0, h1, h2, h3, invd, Wl, Wr, b, Wl3,
             hout, m0, m1):
    agg = jnp.concatenate([a0[...], a1[...], a2[...], a3[...]], axis=1)
    hprev = jnp.concatenate([h0[...], h1[...], h2[...], h3[...]], axis=1)
    h = jnp.maximum(
        jnp.dot(agg * invd[...], Wl[...], preferred_element_type=jnp.float32)
        + jnp.dot(hprev, Wr[...], preferred_element_type=jnp.float32)
        + b[...], 0.0)
    m = jnp.dot(h, Wl3[...], preferred_element_type=jnp.float32)
    hout[...] = h
    m0[...] = m[:, 0:16]
    m1[...] = m[:, 16:32]


def _tc_layer2(accs, hs, invd, Wl2, Wr2, b2, Wl3):
    return pl.pallas_call(
        _l2_body,
        grid=(GRID,),
        in_specs=[_aspec(0), _aspec(1), _aspec(2), _aspec(3)]
        + [_bs(16)] * 4 + [_bs(1)]
        + [_full((64, 64)), _full((64, 64)), _full((1, 64)),
           _full((64, 32))],
        out_specs=[_bs(64), _bs(16), _bs(16)],
        out_shape=[jax.ShapeDtypeStruct((N_P, 64), jnp.float32),
                   jax.ShapeDtypeStruct((N_P, 16), jnp.float32),
                   jax.ShapeDtypeStruct((N_P, 16), jnp.float32)],
    )(accs, accs, accs, accs, *hs, invd, Wl2, Wr2, b2, Wl3)


def _l3_body(a0, a1, h2, invd, Wr, b, W4, b4, W5, b5, out):
    agg = jnp.concatenate([a0[...], a1[...]], axis=1)
    h = jnp.maximum(
        agg * invd[...]
        + jnp.dot(h2[...], Wr[...], preferred_element_type=jnp.float32)
        + b[...], 0.0)
    h = jnp.maximum(
        jnp.dot(h, W4[...], preferred_element_type=jnp.float32)
        + b4[...], 0.0)
    out[...] = (jnp.dot(h, W5[...], preferred_element_type=jnp.float32)
                + b5[...])


def _tc_layer3(accs, h2, invd, Wr3, b3, W4, b4, W5, b5):
    return pl.pallas_call(
        _l3_body,
        grid=(GRID,),
        in_specs=[_aspec(0), _aspec(1), _bs(64), _bs(1),
                  _full((64, 32)), _full((1, 32)), _full((32, 16)),
                  _full((1, 16)), _full((16, 1)), _full((1, 1))],
        out_specs=_bs(1),
        out_shape=jax.ShapeDtypeStruct((N_P, 1), jnp.float32),
    )(accs, accs, h2, invd, Wr3, b3, W4, b4, W5, b5)


def kernel(x, edge_index, Wl1, Wr1, b1, Wl2, Wr2, b2, Wl3, Wr3, b3,
           W4, b4, W5, b5):
    src = edge_index[0].astype(jnp.int32)
    dst = edge_index[1].astype(jnp.int32)
    padi = jnp.full((E_P - N_EDGES,), PAD_NODE, jnp.int32)
    src_rows = jnp.concatenate([src, padi]).reshape(E_P // CH, CH)
    dst_rows = jnp.concatenate([dst, padi]).reshape(E_P // CH, CH)

    ones = jnp.ones((N_NODES, 1), jnp.float32)
    x16 = jnp.pad(jnp.concatenate([x, ones], axis=1),
                  ((0, N_P - N_NODES), (0, 11)))
    x4 = jnp.pad(x, ((0, N_P - N_NODES), (0, 0)))

    acc1 = _seg_sum16([x16], src_rows, dst_rows, [0])
    h0, h1, h2t, h3t, invd = _tc_layer1(acc1, x4, Wl1.astype(jnp.float32),
                                        Wr1, b1.reshape(1, 64))

    a2 = _seg_sum16([h0, h1, h2t, h3t], src_rows, dst_rows, [0, 0, 1, 1])
    h2, m0, m1 = _tc_layer2(a2, (h0, h1, h2t, h3t), invd,
                            Wl2, Wr2, b2.reshape(1, 64), Wl3)

    a3 = _seg_sum16([m0, m1], src_rows, dst_rows, [0, 1])
    out = _tc_layer3(a3, h2, invd, Wr3, b3.reshape(1, 32),
                     W4, b4.reshape(1, 16), W5, b5.reshape(1, 1))
    return out[:N_NODES, 0]


# speedup vs baseline: 1.3456x; 1.0097x over previous
import jax
import jax.numpy as jnp
from jax import lax
from jax.experimental import pallas as pl
from jax.experimental.pallas import tpu as pltpu
from jax.experimental.pallas import tpu_sc as plsc

N_NODES = 100000
N_EDGES = 1600000
PAD_NODE = N_NODES
N_P = 100352
E_P = 1638400
CH = 128
GRP = 4
NC, NS = 2, 16
ROWS_PER_STAGE = 40
GPS = ROWS_PER_STAGE // GRP
ROWS_PER_TILE = E_P // CH // NS
STAGES = ROWS_PER_TILE // ROWS_PER_STAGE
ACC_N = 100016
ACC_ROWS_PER_TILE = ACC_N // NS
NZ = 47
ZCH = ACC_ROWS_PER_TILE // NZ


def _make_seg_body(assign):
  def _seg_body_multi(*refs):
    ntab = len(refs) - 12
    tables = refs[:ntab]
    (src_hbm, dst_hbm, out_hbm, acc_sh, rv0, rv1, svb, dvb, zbuf,
     gsem0, gsem1, ssem) = refs[ntab:]
    c = lax.axis_index("c")
    s = lax.axis_index("s")
    z0 = s * ACC_ROWS_PER_TILE

    def _zfill(i, _):
        zbuf[i, :] = jnp.zeros((16,), jnp.float32)
        return 0
    lax.fori_loop(0, ZCH, _zfill, 0)

    for p, (table, (core, base, nst)) in enumerate(zip(tables, assign)):
      @pl.when(c == core)
      def _panel(p=p, table=table, base=base, nst=nst):
        wbase = base + s * (nst * ROWS_PER_STAGE)
        zc = [pltpu.async_copy(zbuf, acc_sh.at[pl.ds(z0 + k * ZCH, ZCH)],
                               gsem0) for k in range(NZ)]
        for d in zc:
            d.wait()
        plsc.subcore_barrier()

        def _stage(f, _, table=table):
            fb = wbase + f * ROWS_PER_STAGE
            pltpu.sync_copy(src_hbm.at[pl.ds(fb, ROWS_PER_STAGE)], svb)
            pltpu.sync_copy(dst_hbm.at[pl.ds(fb, ROWS_PER_STAGE)], dvb)
            for j in range(GRP):
                pltpu.async_copy(table.at[svb.at[j]], rv0.at[j], gsem0)

            def _pair(i, _):
                r1 = (2 * i + 1) * GRP
                for j in range(GRP):
                    pltpu.async_copy(table.at[svb.at[r1 + j]],
                                     rv1.at[j], gsem1)
                for j in range(GRP):
                    pltpu.make_async_copy(table.at[svb.at[j]],
                                          rv0.at[j], gsem0).wait()
                r0 = (2 * i) * GRP
                sc0 = [pltpu.async_copy(rv0.at[j],
                                        acc_sh.at[dvb.at[r0 + j]],
                                        ssem, add=True)
                       for j in range(GRP)]
                for d in sc0:
                    d.wait()
                r2 = ((2 * i + 2) * GRP) % ROWS_PER_STAGE
                for j in range(GRP):
                    pltpu.async_copy(table.at[svb.at[r2 + j]],
                                     rv0.at[j], gsem0)
                for j in range(GRP):
                    pltpu.make_async_copy(table.at[svb.at[j]],
                                          rv1.at[j], gsem1).wait()
                sc1 = [pltpu.async_copy(rv1.at[j],
                                        acc_sh.at[dvb.at[r1 + j]],
                                        ssem, add=True)
                       for j in range(GRP)]
                for d in sc1:
                    d.wait()
                return 0

            lax.fori_loop(0, GPS // 2, _pair, 0)
            for j in range(GRP):
                pltpu.make_async_copy(table.at[svb.at[j]],
                                      rv0.at[j], gsem0).wait()
            return 0

        lax.fori_loop(0, nst, _stage, 0)
        plsc.subcore_barrier()
        pltpu.sync_copy(acc_sh.at[pl.ds(z0, ACC_ROWS_PER_TILE)],
                        out_hbm.at[p, pl.ds(z0, ACC_ROWS_PER_TILE)])

  return _seg_body_multi


def _seg_sum16(tables, src_rows, dst_rows, assign):
    ntab = len(tables)
    return pl.kernel(
        _make_seg_body(assign),
        out_type=jax.ShapeDtypeStruct((ntab, N_P, 16), jnp.float32),
        mesh=plsc.VectorSubcoreMesh(core_axis_name="c", subcore_axis_name="s"),
        compiler_params=pltpu.CompilerParams(use_tc_tiling_on_sc=False),
        scratch_types=[
            pltpu.VMEM_SHARED((ACC_N, 16), jnp.float32),
            pltpu.VMEM((GRP, CH, 16), jnp.float32),
            pltpu.VMEM((GRP, CH, 16), jnp.float32),
            pltpu.VMEM((ROWS_PER_STAGE, CH), jnp.int32),
            pltpu.VMEM((ROWS_PER_STAGE, CH), jnp.int32),
            pltpu.VMEM((ZCH, 16), jnp.float32),
            pltpu.SemaphoreType.DMA,
            pltpu.SemaphoreType.DMA,
            pltpu.SemaphoreType.DMA,
        ],
    )(*tables, src_rows, dst_rows)


BLK = 2048
GRID = N_P // BLK


def _bs(f):
    return pl.BlockSpec((BLK, f), lambda i: (i, 0))


def _full(shape):
    return pl.BlockSpec(shape, lambda i: tuple(0 for _ in shape))


def _aspec(p):
    return pl.BlockSpec((None, BLK, 16), lambda i, p=p: (p, i, 0))


def _l1_body(a, a1p, x, Wl, Wr, b, h0, h1, h2, h3, invd):
    acc = a[...] + a1p[...]
    inv = 1.0 / jnp.maximum(acc[:, 4:5], 1.0)
    h = jnp.maximum(
        jnp.dot(acc[:, :4] * inv, Wl[...], preferred_element_type=jnp.float32)
        + jnp.dot(x[...], Wr[...], preferred_element_type=jnp.float32)
        + b[...], 0.0)
    h0[...] = h[:, 0:16]
    h1[...] = h[:, 16:32]
    h2[...] = h[:, 32:48]
    h3[...] = h[:, 48:64]
    invd[...] = inv


def _tc_layer1(acc, x4, Wl1, Wr1, b1):
    return pl.pallas_call(
        _l1_body,
        grid=(GRID,),
        in_specs=[_aspec(0), _aspec(1), _bs(4), _full((4, 64)),
                  _full((4, 64)), _full((1, 64))],
        out_specs=[_bs(16), _bs(16), _bs(16), _bs(16), _bs(1)],
        out_shape=[jax.ShapeDtypeStruct((N_P, 16), jnp.float32)] * 4
        + [jax.ShapeDtypeStruct((N_P, 1), jnp.float32)],
    )(acc, acc, x4, Wl1, Wr1, b1)


def _l2_body(a0, a1, a2, a3, h0, h1, h2, h3, invd, Wl, Wr, b, Wl3,
             hout, m0, m1):
    agg = jnp.concatenate([a0[...], a1[...], a2[...], a3[...]], axis=1)
    hprev = jnp.concatenate([h0[...], h1[...], h2[...], h3[...]], axis=1)
    h = jnp.maximum(
        jnp.dot(agg * invd[...], Wl[...], preferred_element_type=jnp.float32)
        + jnp.dot(hprev, Wr[...], preferred_element_type=jnp.float32)
        + b[...], 0.0)
    m = jnp.dot(h, Wl3[...], preferred_element_type=jnp.float32)
    hout[...] = h
    m0[...] = m[:, 0:16]
    m1[...] = m[:, 16:32]


def _tc_layer2(accs, hs, invd, Wl2, Wr2, b2, Wl3):
    return pl.pallas_call(
        _l2_body,
        grid=(GRID,),
        in_specs=[_aspec(0), _aspec(1), _aspec(2), _aspec(3)]
        + [_bs(16)] * 4 + [_bs(1)]
        + [_full((64, 64)), _full((64, 64)), _full((1, 64)),
           _full((64, 32))],
        out_specs=[_bs(64), _bs(16), _bs(16)],
        out_shape=[jax.ShapeDtypeStruct((N_P, 64), jnp.float32),
                   jax.ShapeDtypeStruct((N_P, 16), jnp.float32),
                   jax.ShapeDtypeStruct((N_P, 16), jnp.float32)],
    )(accs, accs, accs, accs, *hs, invd, Wl2, Wr2, b2, Wl3)


def _l3_body(a0, a1, h2, invd, Wr, b, W4, b4, W5, b5, out):
    agg = jnp.concatenate([a0[...], a1[...]], axis=1)
    h = jnp.maximum(
        agg * invd[...]
        + jnp.dot(h2[...], Wr[...], preferred_element_type=jnp.float32)
        + b[...], 0.0)
    h = jnp.maximum(
        jnp.dot(h, W4[...], preferred_element_type=jnp.float32)
        + b4[...], 0.0)
    out[...] = (jnp.dot(h, W5[...], preferred_element_type=jnp.float32)
                + b5[...])


def _tc_layer3(accs, h2, invd, Wr3, b3, W4, b4, W5, b5):
    return pl.pallas_call(
        _l3_body,
        grid=(GRID,),
        in_specs=[_aspec(0), _aspec(1), _bs(64), _bs(1),
                  _full((64, 32)), _full((1, 32)), _full((32, 16)),
                  _full((1, 16)), _full((16, 1)), _full((1, 1))],
        out_specs=_bs(1),
        out_shape=jax.ShapeDtypeStruct((N_P, 1), jnp.float32),
    )(accs, accs, h2, invd, Wr3, b3, W4, b4, W5, b5)


def kernel(x, edge_index, Wl1, Wr1, b1, Wl2, Wr2, b2, Wl3, Wr3, b3,
           W4, b4, W5, b5):
    src = edge_index[0].astype(jnp.int32)
    dst = edge_index[1].astype(jnp.int32)
    padi = jnp.full((E_P - N_EDGES,), PAD_NODE, jnp.int32)
    src_rows = jnp.concatenate([src, padi]).reshape(E_P // CH, CH)
    dst_rows = jnp.concatenate([dst, padi]).reshape(E_P // CH, CH)

    ones = jnp.ones((N_NODES, 1), jnp.float32)
    x16 = jnp.pad(jnp.concatenate([x, ones], axis=1),
                  ((0, N_P - N_NODES), (0, 11)))
    x4 = jnp.pad(x, ((0, N_P - N_NODES), (0, 0)))

    acc1 = _seg_sum16([x16, x16], src_rows, dst_rows,
                      [(0, 0, 15), (1, 9600, 5)])
    h0, h1, h2t, h3t, invd = _tc_layer1(acc1, x4, Wl1.astype(jnp.float32),
                                        Wr1, b1.reshape(1, 64))

    a2 = _seg_sum16([h0, h1, h2t, h3t], src_rows, dst_rows,
                    [(0, 0, 20), (0, 0, 20), (1, 0, 20), (1, 0, 20)])
    h2, m0, m1 = _tc_layer2(a2, (h0, h1, h2t, h3t), invd,
                            Wl2, Wr2, b2.reshape(1, 64), Wl3)

    a3 = _seg_sum16([m0, m1], src_rows, dst_rows,
                    [(0, 0, 20), (1, 0, 20)])
    out = _tc_layer3(a3, h2, invd, Wr3, b3.reshape(1, 32),
                     W4, b4.reshape(1, 16), W5, b5.reshape(1, 1))
    return out[:N_NODES, 0]
